# Initial kernel scaffold; baseline (speedup 1.0000x reference)
import functools

import jax
import jax.numpy as jnp
from jax import lax
from jax.experimental import pallas as pl
from jax.experimental.pallas import tpu as pltpu
from jax.experimental.pallas import tpu_sc as plsc

N = 10000
E = 160000
DF = 128
DE = 4
LAT = 8
EMB = 64
G = 64

NC = 2
NS = 16
NW = NC * NS
EPW = 5120
CHUNK = 1024
NCHUNK = EPW // CHUNK
E_PAD = NW * EPW
N_ACC = 10240
ROWS_PER_TILE = N_ACC // NS

XPW = 80

NB = 2000
EB = 2048


def _prep_body(x_ref, w_ref, o_ref):
    o_ref[...] = jnp.dot(x_ref[...], w_ref[...],
                         preferred_element_type=jnp.float32)


def _combine_body(xpg_ref, ea_ref, w1_ref, b1_ref, r_ref, s_ref, o_ref):
    h = jnp.maximum(
        jnp.dot(ea_ref[...], w1_ref[...],
                preferred_element_type=jnp.float32) + b1_ref[...], 0.0)
    hexp = jnp.dot(h, r_ref[...], preferred_element_type=jnp.float32)
    xpg = xpg_ref[...]
    o_ref[...] = jnp.dot(xpg[:, :64] * hexp, s_ref[...],
                         preferred_element_type=jnp.float32) + xpg[:, 64:72]


def _post_body(p0_ref, p1_ref, xp_ref, b3_ref, broot_ref, wfc_ref, bfc_ref,
               o_ref, sums_ref, counts_ref):
    i = pl.program_id(0)

    @pl.when(i == 0)
    def _():
        sums_ref[...] = jnp.zeros_like(sums_ref)
        counts_ref[...] = jnp.zeros_like(counts_ref)

    outb = jnp.maximum(
        p0_ref[...] + p1_ref[...] + xp_ref[:, 72:80] + broot_ref[...], 0.0)
    bb = b3_ref[0]
    gi = lax.broadcasted_iota(jnp.int32, (G, NB), 0)
    og = (bb == gi).astype(jnp.float32)
    sums_ref[...] += jnp.dot(og, outb, preferred_element_type=jnp.float32)
    counts_ref[...] += jnp.dot(og, jnp.ones_like(outb),
                               preferred_element_type=jnp.float32)

    @pl.when(i == pl.num_programs(0) - 1)
    def _():
        pooled = sums_ref[...] / jnp.maximum(counts_ref[...], 1.0)
        o_ref[...] = jnp.dot(jnp.maximum(pooled, 0.0), wfc_ref[...],
                             preferred_element_type=jnp.float32) + bfc_ref[...]


def _sc_gather(xp, src2d):
    mesh = plsc.VectorSubcoreMesh(core_axis_name="c", subcore_axis_name="s")

    @functools.partial(
        pl.kernel,
        out_type=jax.ShapeDtypeStruct((E_PAD, XPW), jnp.float32),
        mesh=mesh,
        scratch_types=[
            pltpu.VMEM((8, 128), jnp.int32),
            pltpu.VMEM((CHUNK, XPW), jnp.float32),
            pltpu.SemaphoreType.DMA,
        ],
    )
    def gk(xp_hbm, src_hbm, out_hbm, idx_v, rows_v, sem):
        c = lax.axis_index("c")
        s = lax.axis_index("s")
        wid = s * NC + c

        def body(ci, carry):
            base = wid * EPW + ci * CHUNK
            pltpu.sync_copy(src_hbm.at[pl.ds(wid * (EPW // 128) + ci * 8, 8)],
                            idx_v)
            cps = [pltpu.async_copy(xp_hbm.at[idx_v.at[j]],
                                    rows_v.at[pl.ds(j * 128, 128)], sem)
                   for j in range(8)]
            for cp in cps:
                cp.wait()
            pltpu.sync_copy(rows_v, out_hbm.at[pl.ds(base, CHUNK)])
            return carry

        lax.fori_loop(0, NCHUNK, body, 0)

    return gk(xp, src2d)


def _sc_scatter(msg, dst2d, zrows):
    mesh = plsc.VectorSubcoreMesh(core_axis_name="c", subcore_axis_name="s")

    @functools.partial(
        pl.kernel,
        out_type=jax.ShapeDtypeStruct((NC * N_ACC, LAT), jnp.float32),
        mesh=mesh,
        scratch_types=[
            pltpu.VMEM((8, 128), jnp.int32),
            pltpu.VMEM((CHUNK, LAT), jnp.float32),
            pltpu.VMEM_SHARED((N_ACC, LAT), jnp.float32),
        ],
    )
    def sk(msg_hbm, dst_hbm, z_hbm, out_hbm, idx_v, rows_v, acc):
        c = lax.axis_index("c")
        s = lax.axis_index("s")
        wid = s * NC + c
        pltpu.sync_copy(z_hbm.at[pl.ds(s * ROWS_PER_TILE, ROWS_PER_TILE)],
                        acc.at[pl.ds(s * ROWS_PER_TILE, ROWS_PER_TILE)])
        plsc.subcore_barrier()

        def body(ci, carry):
            base = wid * EPW + ci * CHUNK
            pltpu.sync_copy(dst_hbm.at[pl.ds(wid * (EPW // 128) + ci * 8, 8)],
                            idx_v)
            pltpu.sync_copy(msg_hbm.at[pl.ds(base, CHUNK)], rows_v)
            for j in range(8):
                pltpu.sync_copy(rows_v.at[pl.ds(j * 128, 128)],
                                acc.at[idx_v.at[j]], add=True)
            return carry

        lax.fori_loop(0, NCHUNK, body, 0)
        plsc.subcore_barrier()
        pltpu.sync_copy(
            acc.at[pl.ds(s * ROWS_PER_TILE, ROWS_PER_TILE)],
            out_hbm.at[pl.ds(c * N_ACC + s * ROWS_PER_TILE, ROWS_PER_TILE)])

    return sk(msg, dst2d, zrows)


def kernel(x, edge_index, edge_attr, batch, W1, b1, W2, b2,
           W_root, b_root, Wfc, bfc):
    src = edge_index[0].astype(jnp.int32)
    dst = edge_index[1].astype(jnp.int32)

    w2r = W2.reshape(LAT, DF, LAT).transpose(1, 0, 2).reshape(DF, LAT * LAT)
    wcat = jnp.concatenate([w2r, b2.reshape(DF, LAT), W_root], axis=1)

    r_mat = (jnp.arange(LAT * LAT)[None, :] // LAT
             == jnp.arange(LAT)[:, None]).astype(jnp.float32)
    s_mat = (jnp.arange(LAT * LAT)[:, None] % LAT
             == jnp.arange(LAT)[None, :]).astype(jnp.float32)

    pad = E_PAD - E
    src_p = jnp.concatenate([src, jnp.zeros((pad,), jnp.int32)])
    dst_p = jnp.concatenate([dst, jnp.full((pad,), N, jnp.int32)])
    ea_p = jnp.concatenate([edge_attr, jnp.zeros((pad, DE), jnp.float32)])
    src2d = src_p.reshape(E_PAD // 128, 128)
    dst2d = dst_p.reshape(E_PAD // 128, 128)
    zrows = jnp.zeros((N_ACC, LAT), jnp.float32)
    batch3 = batch.astype(jnp.int32).reshape(N // NB, 1, NB)

    xp = pl.pallas_call(
        _prep_body,
        grid=(N // NB,),
        in_specs=[
            pl.BlockSpec((NB, DF), lambda i: (i, 0)),
            pl.BlockSpec((DF, XPW), lambda i: (0, 0)),
        ],
        out_specs=pl.BlockSpec((NB, XPW), lambda i: (i, 0)),
        out_shape=jax.ShapeDtypeStruct((N, XPW), jnp.float32),
    )(x, wcat)

    xpg = _sc_gather(xp, src2d)

    msg = pl.pallas_call(
        _combine_body,
        grid=(E_PAD // EB,),
        in_specs=[
            pl.BlockSpec((EB, XPW), lambda i: (i, 0)),
            pl.BlockSpec((EB, DE), lambda i: (i, 0)),
            pl.BlockSpec((DE, LAT), lambda i: (0, 0)),
            pl.BlockSpec((1, LAT), lambda i: (0, 0)),
            pl.BlockSpec((LAT, LAT * LAT), lambda i: (0, 0)),
            pl.BlockSpec((LAT * LAT, LAT), lambda i: (0, 0)),
        ],
        out_specs=pl.BlockSpec((EB, LAT), lambda i: (i, 0)),
        out_shape=jax.ShapeDtypeStruct((E_PAD, LAT), jnp.float32),
    )(xpg, ea_p, W1, b1.reshape(1, LAT), r_mat, s_mat)

    p01 = _sc_scatter(msg, dst2d, zrows)
    p0 = p01[:N_ACC]
    p1 = p01[N_ACC:]

    res = pl.pallas_call(
        _post_body,
        grid=(N // NB,),
        in_specs=[
            pl.BlockSpec((NB, LAT), lambda i: (i, 0)),
            pl.BlockSpec((NB, LAT), lambda i: (i, 0)),
            pl.BlockSpec((NB, XPW), lambda i: (i, 0)),
            pl.BlockSpec((1, 1, NB), lambda i: (i, 0, 0)),
            pl.BlockSpec((1, LAT), lambda i: (0, 0)),
            pl.BlockSpec((LAT, EMB), lambda i: (0, 0)),
            pl.BlockSpec((1, EMB), lambda i: (0, 0)),
        ],
        out_specs=pl.BlockSpec((G, EMB), lambda i: (0, 0)),
        out_shape=jax.ShapeDtypeStruct((G, EMB), jnp.float32),
        scratch_shapes=[
            pltpu.VMEM((G, LAT), jnp.float32),
            pltpu.VMEM((G, LAT), jnp.float32),
        ],
    )(p0, p1, xp, batch3, b_root.reshape(1, LAT), Wfc, bfc.reshape(1, EMB))

    return res


# speedup vs baseline: 2.6912x; 2.6912x over previous
import functools

import jax
import jax.numpy as jnp
from jax import lax
from jax.experimental import pallas as pl
from jax.experimental.pallas import tpu as pltpu
from jax.experimental.pallas import tpu_sc as plsc

N = 10000
E = 160000
DF = 128
DE = 4
LAT = 8
EMB = 64
G = 64

NC = 2
NS = 16
NW = NC * NS
EPW = 5120
CHUNK = 1024
NCHUNK = EPW // CHUNK
E_PAD = NW * EPW
N_ACC = 10240
ROWS_PER_TILE = N_ACC // NS

XPW = 80

NB = 2000
EB = 2048


def _prep_body(x_ref, w_ref, o_ref):
    o_ref[...] = jnp.dot(x_ref[...], w_ref[...],
                         preferred_element_type=jnp.float32)


def _combine_body(xpg_ref, ea_ref, w1_ref, b1_ref, r_ref, s_ref, o_ref):
    h = jnp.maximum(
        jnp.dot(ea_ref[...], w1_ref[...],
                preferred_element_type=jnp.float32) + b1_ref[...], 0.0)
    hexp = jnp.dot(h, r_ref[...], preferred_element_type=jnp.float32)
    xpg = xpg_ref[...]
    o_ref[...] = jnp.dot(xpg[:, :64] * hexp, s_ref[...],
                         preferred_element_type=jnp.float32) + xpg[:, 64:72]


def _post_body(p0_ref, p1_ref, xp_ref, b3_ref, broot_ref, wfc_ref, bfc_ref,
               o_ref, sums_ref, counts_ref):
    i = pl.program_id(0)

    @pl.when(i == 0)
    def _():
        sums_ref[...] = jnp.zeros_like(sums_ref)
        counts_ref[...] = jnp.zeros_like(counts_ref)

    outb = jnp.maximum(
        p0_ref[...] + p1_ref[...] + xp_ref[:, 72:80] + broot_ref[...], 0.0)
    bb = b3_ref[0]
    gi = lax.broadcasted_iota(jnp.int32, (G, NB), 0)
    og = (bb == gi).astype(jnp.float32)
    sums_ref[...] += jnp.dot(og, outb, preferred_element_type=jnp.float32)
    counts_ref[...] += jnp.dot(og, jnp.ones_like(outb),
                               preferred_element_type=jnp.float32)

    @pl.when(i == pl.num_programs(0) - 1)
    def _():
        pooled = sums_ref[...] / jnp.maximum(counts_ref[...], 1.0)
        o_ref[...] = jnp.dot(jnp.maximum(pooled, 0.0), wfc_ref[...],
                             preferred_element_type=jnp.float32) + bfc_ref[...]


def _sc_gather(xp, src2d):
    mesh = plsc.VectorSubcoreMesh(core_axis_name="c", subcore_axis_name="s")

    @functools.partial(
        pl.kernel,
        out_type=jax.ShapeDtypeStruct((E_PAD, XPW), jnp.float32),
        mesh=mesh,
        scratch_types=[
            pltpu.VMEM((8, 128), jnp.int32),
            pltpu.VMEM((CHUNK, XPW), jnp.float32),
            pltpu.SemaphoreType.DMA,
        ],
        compiler_params=pltpu.CompilerParams(use_tc_tiling_on_sc=False),
    )
    def gk(xp_hbm, src_hbm, out_hbm, idx_v, rows_v, sem):
        c = lax.axis_index("c")
        s = lax.axis_index("s")
        wid = s * NC + c

        def body(ci, carry):
            base = wid * EPW + ci * CHUNK
            pltpu.sync_copy(src_hbm.at[pl.ds(wid * (EPW // 128) + ci * 8, 8)],
                            idx_v)
            cps = [pltpu.async_copy(xp_hbm.at[idx_v.at[j]],
                                    rows_v.at[pl.ds(j * 128, 128)], sem)
                   for j in range(8)]
            for cp in cps:
                cp.wait()
            pltpu.sync_copy(rows_v, out_hbm.at[pl.ds(base, CHUNK)])
            return carry

        lax.fori_loop(0, NCHUNK, body, 0)

    return gk(xp, src2d)


def _sc_scatter(msg, dst2d, zrows):
    mesh = plsc.VectorSubcoreMesh(core_axis_name="c", subcore_axis_name="s")

    @functools.partial(
        pl.kernel,
        out_type=jax.ShapeDtypeStruct((NC * N_ACC, LAT), jnp.float32),
        mesh=mesh,
        scratch_types=[
            pltpu.VMEM((8, 128), jnp.int32),
            pltpu.VMEM((CHUNK, LAT), jnp.float32),
            pltpu.VMEM_SHARED((N_ACC, LAT), jnp.float32),
        ],
        compiler_params=pltpu.CompilerParams(use_tc_tiling_on_sc=False),
    )
    def sk(msg_hbm, dst_hbm, z_hbm, out_hbm, idx_v, rows_v, acc):
        c = lax.axis_index("c")
        s = lax.axis_index("s")
        wid = s * NC + c
        pltpu.sync_copy(z_hbm.at[pl.ds(s * ROWS_PER_TILE, ROWS_PER_TILE)],
                        acc.at[pl.ds(s * ROWS_PER_TILE, ROWS_PER_TILE)])
        plsc.subcore_barrier()

        def body(ci, carry):
            base = wid * EPW + ci * CHUNK
            pltpu.sync_copy(dst_hbm.at[pl.ds(wid * (EPW // 128) + ci * 8, 8)],
                            idx_v)
            pltpu.sync_copy(msg_hbm.at[pl.ds(base, CHUNK)], rows_v)
            for j in range(8):
                pltpu.sync_copy(rows_v.at[pl.ds(j * 128, 128)],
                                acc.at[idx_v.at[j]], add=True)
            return carry

        lax.fori_loop(0, NCHUNK, body, 0)
        plsc.subcore_barrier()
        pltpu.sync_copy(
            acc.at[pl.ds(s * ROWS_PER_TILE, ROWS_PER_TILE)],
            out_hbm.at[pl.ds(c * N_ACC + s * ROWS_PER_TILE, ROWS_PER_TILE)])

    return sk(msg, dst2d, zrows)


def kernel(x, edge_index, edge_attr, batch, W1, b1, W2, b2,
           W_root, b_root, Wfc, bfc):
    src = edge_index[0].astype(jnp.int32)
    dst = edge_index[1].astype(jnp.int32)

    w2r = W2.reshape(LAT, DF, LAT).transpose(1, 0, 2).reshape(DF, LAT * LAT)
    wcat = jnp.concatenate([w2r, b2.reshape(DF, LAT), W_root], axis=1)

    r_mat = (jnp.arange(LAT * LAT)[None, :] // LAT
             == jnp.arange(LAT)[:, None]).astype(jnp.float32)
    s_mat = (jnp.arange(LAT * LAT)[:, None] % LAT
             == jnp.arange(LAT)[None, :]).astype(jnp.float32)

    pad = E_PAD - E
    src_p = jnp.concatenate([src, jnp.zeros((pad,), jnp.int32)])
    dst_p = jnp.concatenate([dst, jnp.full((pad,), N, jnp.int32)])
    ea_p = jnp.concatenate([edge_attr, jnp.zeros((pad, DE), jnp.float32)])
    src2d = src_p.reshape(E_PAD // 128, 128)
    dst2d = dst_p.reshape(E_PAD // 128, 128)
    zrows = jnp.zeros((N_ACC, LAT), jnp.float32)
    batch3 = batch.astype(jnp.int32).reshape(N // NB, 1, NB)

    xp = pl.pallas_call(
        _prep_body,
        grid=(N // NB,),
        in_specs=[
            pl.BlockSpec((NB, DF), lambda i: (i, 0)),
            pl.BlockSpec((DF, XPW), lambda i: (0, 0)),
        ],
        out_specs=pl.BlockSpec((NB, XPW), lambda i: (i, 0)),
        out_shape=jax.ShapeDtypeStruct((N, XPW), jnp.float32),
    )(x, wcat)

    xpg = _sc_gather(xp, src2d)

    msg = pl.pallas_call(
        _combine_body,
        grid=(E_PAD // EB,),
        in_specs=[
            pl.BlockSpec((EB, XPW), lambda i: (i, 0)),
            pl.BlockSpec((EB, DE), lambda i: (i, 0)),
            pl.BlockSpec((DE, LAT), lambda i: (0, 0)),
            pl.BlockSpec((1, LAT), lambda i: (0, 0)),
            pl.BlockSpec((LAT, LAT * LAT), lambda i: (0, 0)),
            pl.BlockSpec((LAT * LAT, LAT), lambda i: (0, 0)),
        ],
        out_specs=pl.BlockSpec((EB, LAT), lambda i: (i, 0)),
        out_shape=jax.ShapeDtypeStruct((E_PAD, LAT), jnp.float32),
    )(xpg, ea_p, W1, b1.reshape(1, LAT), r_mat, s_mat)

    p01 = _sc_scatter(msg, dst2d, zrows)
    p0 = p01[:N_ACC]
    p1 = p01[N_ACC:]

    res = pl.pallas_call(
        _post_body,
        grid=(N // NB,),
        in_specs=[
            pl.BlockSpec((NB, LAT), lambda i: (i, 0)),
            pl.BlockSpec((NB, LAT), lambda i: (i, 0)),
            pl.BlockSpec((NB, XPW), lambda i: (i, 0)),
            pl.BlockSpec((1, 1, NB), lambda i: (i, 0, 0)),
            pl.BlockSpec((1, LAT), lambda i: (0, 0)),
            pl.BlockSpec((LAT, EMB), lambda i: (0, 0)),
            pl.BlockSpec((1, EMB), lambda i: (0, 0)),
        ],
        out_specs=pl.BlockSpec((G, EMB), lambda i: (0, 0)),
        out_shape=jax.ShapeDtypeStruct((G, EMB), jnp.float32),
        scratch_shapes=[
            pltpu.VMEM((G, LAT), jnp.float32),
            pltpu.VMEM((G, LAT), jnp.float32),
        ],
    )(p0, p1, xp, batch3, b_root.reshape(1, LAT), Wfc, bfc.reshape(1, EMB))

    return res


# speedup vs baseline: 3.0625x; 1.1380x over previous
import functools

import jax
import jax.numpy as jnp
from jax import lax
from jax.experimental import pallas as pl
from jax.experimental.pallas import tpu as pltpu
from jax.experimental.pallas import tpu_sc as plsc

N = 10000
E = 160000
DF = 128
DE = 4
LAT = 8
EMB = 64
G = 64

NC = 2
NS = 16
NW = NC * NS
EPW = 5120
E_PAD = NW * EPW
GCH = 512
NGCH = EPW // GCH
SCH = 1024
NSCH = EPW // SCH
N_ACC = 10240
ROWS_PER_TILE = N_ACC // NS

XPW = 80

NB = 2000
EB = 2048


def _prep_body(x_ref, w_ref, o_ref):
    o_ref[...] = jnp.dot(x_ref[...], w_ref[...],
                         preferred_element_type=jnp.float32)


def _hmat_body(ea_ref, w1_ref, b1_ref, o_ref):
    o_ref[...] = jnp.maximum(
        jnp.dot(ea_ref[...], w1_ref[...],
                preferred_element_type=jnp.float32) + b1_ref[...], 0.0)


def _post_body(p0_ref, p1_ref, xp_ref, b3_ref, broot_ref, wfc_ref, bfc_ref,
               o_ref, sums_ref, counts_ref):
    i = pl.program_id(0)

    @pl.when(i == 0)
    def _():
        sums_ref[...] = jnp.zeros_like(sums_ref)
        counts_ref[...] = jnp.zeros_like(counts_ref)

    outb = jnp.maximum(
        p0_ref[...] + p1_ref[...] + xp_ref[:, 72:80] + broot_ref[...], 0.0)
    bb = b3_ref[0]
    gi = lax.broadcasted_iota(jnp.int32, (G, NB), 0)
    og = (bb == gi).astype(jnp.float32)
    sums_ref[...] += jnp.dot(og, outb, preferred_element_type=jnp.float32)
    counts_ref[...] += jnp.dot(og, jnp.ones_like(outb),
                               preferred_element_type=jnp.float32)

    @pl.when(i == pl.num_programs(0) - 1)
    def _():
        pooled = sums_ref[...] / jnp.maximum(counts_ref[...], 1.0)
        o_ref[...] = jnp.dot(jnp.maximum(pooled, 0.0), wfc_ref[...],
                             preferred_element_type=jnp.float32) + bfc_ref[...]


def _vgather(v, idx):
    dnums = lax.GatherDimensionNumbers(
        offset_dims=(), collapsed_slice_dims=(0,), start_index_map=(0,))
    return lax.gather(v, idx[:, None], dnums, slice_sizes=(1,),
                      mode=lax.GatherScatterMode.PROMISE_IN_BOUNDS)


def _sc_gather_combine(xp, src2d, hpair):
    mesh = plsc.VectorSubcoreMesh(core_axis_name="c", subcore_axis_name="s")

    @functools.partial(
        pl.kernel,
        out_type=jax.ShapeDtypeStruct((E_PAD // 2, 16), jnp.float32),
        mesh=mesh,
        scratch_types=[
            pltpu.VMEM((2, 4, 128), jnp.int32),
            pltpu.VMEM((2, GCH, XPW), jnp.float32),
            pltpu.VMEM((2, GCH // 2, 16), jnp.float32),
            pltpu.VMEM((2, GCH // 2, 16), jnp.float32),
            pltpu.SemaphoreType.DMA,
            pltpu.SemaphoreType.DMA,
            pltpu.SemaphoreType.DMA,
            pltpu.SemaphoreType.DMA,
            pltpu.SemaphoreType.DMA,
        ],
        compiler_params=pltpu.CompilerParams(use_tc_tiling_on_sc=False),
    )
    def gk(xp_hbm, src_hbm, hp_hbm, out_hbm, idx_v, rows_v, h_v, msg_v,
           gsem0, gsem1, hsem0, hsem1, osem):
        c = lax.axis_index("c")
        s = lax.axis_index("s")
        wid = s * NC + c
        gsems = (gsem0, gsem1)
        hsems = (hsem0, hsem1)

        def issue(ci, b):
            pltpu.sync_copy(
                src_hbm.at[pl.ds(wid * (EPW // 128) + ci * 4, 4)],
                idx_v.at[b])
            cps = [pltpu.async_copy(xp_hbm.at[idx_v.at[b].at[j]],
                                    rows_v.at[b].at[pl.ds(j * 128, 128)],
                                    gsems[b])
                   for j in range(4)]
            hcp = pltpu.async_copy(
                hp_hbm.at[pl.ds(wid * (EPW // 2) + ci * (GCH // 2),
                                GCH // 2)],
                h_v.at[b], hsems[b])
            return cps, hcp

        iota = lax.broadcasted_iota(jnp.int32, (16,), 0)
        swp = iota ^ 8
        lo = iota < 8

        def compute(b):
            rows = rows_v.at[b]
            hrows = h_v.at[b]
            mrows = msg_v.at[b]

            def pair(p, carry):
                hv = hrows[p, :]
                acc_a = jnp.zeros((16,), jnp.float32)
                acc_b = jnp.zeros((16,), jnp.float32)
                for r in range(4):
                    ia = jnp.where(lo, r, r + 4)
                    va = rows[2 * p, pl.ds(16 * r, 16)]
                    vb = rows[2 * p + 1, pl.ds(16 * r, 16)]
                    acc_a = acc_a + va * _vgather(hv, ia)
                    acc_b = acc_b + vb * _vgather(hv, ia + 8)
                macc_a = acc_a + _vgather(acc_a, swp)
                macc_b = acc_b + _vgather(acc_b, swp)
                bias_a = rows[2 * p, pl.ds(64, 16)]
                bias_b = rows[2 * p + 1, pl.ds(64, 16)]
                bias = jnp.where(lo, bias_a, _vgather(bias_b, swp))
                mrows[p, :] = jnp.where(lo, macc_a, macc_b) + bias
                return carry

            lax.fori_loop(0, GCH // 2, pair, 0)

        pending = issue(0, 0)
        out_cp = None
        for ci in range(NGCH):
            b = ci % 2
            cps, hcp = pending
            if ci + 1 < NGCH:
                pending = issue(ci + 1, 1 - b)
            for cp in cps:
                cp.wait()
            hcp.wait()
            if out_cp is not None:
                out_cp.wait()
            compute(b)
            out_cp = pltpu.async_copy(
                msg_v.at[b],
                out_hbm.at[pl.ds(wid * (EPW // 2) + ci * (GCH // 2),
                                 GCH // 2)],
                osem)
        out_cp.wait()

    return gk(xp, src2d, hpair)


def _sc_scatter(msg, dst2d, zrows):
    mesh = plsc.VectorSubcoreMesh(core_axis_name="c", subcore_axis_name="s")

    @functools.partial(
        pl.kernel,
        out_type=jax.ShapeDtypeStruct((NC * N_ACC, LAT), jnp.float32),
        mesh=mesh,
        scratch_types=[
            pltpu.VMEM((8, 128), jnp.int32),
            pltpu.VMEM((SCH, LAT), jnp.float32),
            pltpu.VMEM_SHARED((N_ACC, LAT), jnp.float32),
        ],
        compiler_params=pltpu.CompilerParams(use_tc_tiling_on_sc=False),
    )
    def sk(msg_hbm, dst_hbm, z_hbm, out_hbm, idx_v, rows_v, acc):
        c = lax.axis_index("c")
        s = lax.axis_index("s")
        wid = s * NC + c
        pltpu.sync_copy(z_hbm.at[pl.ds(s * ROWS_PER_TILE, ROWS_PER_TILE)],
                        acc.at[pl.ds(s * ROWS_PER_TILE, ROWS_PER_TILE)])
        plsc.subcore_barrier()

        def body(ci, carry):
            base = wid * EPW + ci * SCH
            pltpu.sync_copy(dst_hbm.at[pl.ds(wid * (EPW // 128) + ci * 8, 8)],
                            idx_v)
            pltpu.sync_copy(msg_hbm.at[pl.ds(base, SCH)], rows_v)
            for j in range(8):
                pltpu.sync_copy(rows_v.at[pl.ds(j * 128, 128)],
                                acc.at[idx_v.at[j]], add=True)
            return carry

        lax.fori_loop(0, NSCH, body, 0)
        plsc.subcore_barrier()
        pltpu.sync_copy(
            acc.at[pl.ds(s * ROWS_PER_TILE, ROWS_PER_TILE)],
            out_hbm.at[pl.ds(c * N_ACC + s * ROWS_PER_TILE, ROWS_PER_TILE)])

    return sk(msg, dst2d, zrows)


def kernel(x, edge_index, edge_attr, batch, W1, b1, W2, b2,
           W_root, b_root, Wfc, bfc):
    src = edge_index[0].astype(jnp.int32)
    dst = edge_index[1].astype(jnp.int32)

    w2r = W2.reshape(LAT, DF, LAT).transpose(1, 0, 2).reshape(DF, LAT * LAT)
    cc = jnp.arange(LAT * LAT)
    rr, ll = cc // 16, cc % 16
    kk = jnp.where(ll < 8, rr, rr + 4)
    perm = kk * LAT + (ll % 8)
    w2p = w2r[:, perm]
    wcat = jnp.concatenate([w2p, b2.reshape(DF, LAT), W_root], axis=1)

    pad = E_PAD - E
    src_p = jnp.concatenate([src, jnp.zeros((pad,), jnp.int32)])
    dst_p = jnp.concatenate([dst, jnp.full((pad,), N, jnp.int32)])
    ea_p = jnp.concatenate([edge_attr, jnp.zeros((pad, DE), jnp.float32)])
    src2d = src_p.reshape(E_PAD // 128, 128)
    dst2d = dst_p.reshape(E_PAD // 128, 128)
    zrows = jnp.zeros((N_ACC, LAT), jnp.float32)
    batch3 = batch.astype(jnp.int32).reshape(N // NB, 1, NB)

    xp = pl.pallas_call(
        _prep_body,
        grid=(N // NB,),
        in_specs=[
            pl.BlockSpec((NB, DF), lambda i: (i, 0)),
            pl.BlockSpec((DF, XPW), lambda i: (0, 0)),
        ],
        out_specs=pl.BlockSpec((NB, XPW), lambda i: (i, 0)),
        out_shape=jax.ShapeDtypeStruct((N, XPW), jnp.float32),
    )(x, wcat)

    hmat = pl.pallas_call(
        _hmat_body,
        grid=(E_PAD // EB,),
        in_specs=[
            pl.BlockSpec((EB, DE), lambda i: (i, 0)),
            pl.BlockSpec((DE, LAT), lambda i: (0, 0)),
            pl.BlockSpec((1, LAT), lambda i: (0, 0)),
        ],
        out_specs=pl.BlockSpec((EB, LAT), lambda i: (i, 0)),
        out_shape=jax.ShapeDtypeStruct((E_PAD, LAT), jnp.float32),
    )(ea_p, W1, b1.reshape(1, LAT))
    hpair = hmat.reshape(E_PAD // 2, 16)

    msg2 = _sc_gather_combine(xp, src2d, hpair)
    msg = msg2.reshape(E_PAD, LAT)

    p01 = _sc_scatter(msg, dst2d, zrows)
    p0 = p01[:N_ACC]
    p1 = p01[N_ACC:]

    res = pl.pallas_call(
        _post_body,
        grid=(N // NB,),
        in_specs=[
            pl.BlockSpec((NB, LAT), lambda i: (i, 0)),
            pl.BlockSpec((NB, LAT), lambda i: (i, 0)),
            pl.BlockSpec((NB, XPW), lambda i: (i, 0)),
            pl.BlockSpec((1, 1, NB), lambda i: (i, 0, 0)),
            pl.BlockSpec((1, LAT), lambda i: (0, 0)),
            pl.BlockSpec((LAT, EMB), lambda i: (0, 0)),
            pl.BlockSpec((1, EMB), lambda i: (0, 0)),
        ],
        out_specs=pl.BlockSpec((G, EMB), lambda i: (0, 0)),
        out_shape=jax.ShapeDtypeStruct((G, EMB), jnp.float32),
        scratch_shapes=[
            pltpu.VMEM((G, LAT), jnp.float32),
            pltpu.VMEM((G, LAT), jnp.float32),
        ],
    )(p0, p1, xp, batch3, b_root.reshape(1, LAT), Wfc, bfc.reshape(1, EMB))

    return res


# speedup vs baseline: 3.3809x; 1.1040x over previous
import functools

import jax
import jax.numpy as jnp
from jax import lax
from jax.experimental import pallas as pl
from jax.experimental.pallas import tpu as pltpu
from jax.experimental.pallas import tpu_sc as plsc

N = 10000
E = 160000
DF = 128
DE = 4
LAT = 8
EMB = 64
G = 64

NC = 2
NS = 16
NW = NC * NS
EPW = 5120
E_PAD = NW * EPW
GCH = 512
NGCH = EPW // GCH
SCH = 1024
NSCH = EPW // SCH
N_ACC = 16000
ROWS_PER_TILE = N_ACC // NS

XPW = 80

NB = 1000


def _prep_body(x_ref, w_ref, ea2_ref, w1p_ref, b1p_ref, xp_ref, h2_ref):
    xp_ref[...] = jnp.dot(x_ref[...], w_ref[...],
                          preferred_element_type=jnp.float32)
    h2_ref[...] = jnp.maximum(
        jnp.dot(ea2_ref[...], w1p_ref[...],
                preferred_element_type=jnp.float32) + b1p_ref[...], 0.0)


def _post_body(p0_ref, p1_ref, xp_ref, b3_ref, broot_ref, wfc_ref, bfc_ref,
               o_ref, sums_ref, counts_ref):
    i = pl.program_id(0)

    @pl.when(i == 0)
    def _():
        sums_ref[...] = jnp.zeros_like(sums_ref)
        counts_ref[...] = jnp.zeros_like(counts_ref)

    outb = jnp.maximum(
        p0_ref[...] + p1_ref[...] + xp_ref[:, 72:80] + broot_ref[...], 0.0)
    bb = b3_ref[0]
    gi = lax.broadcasted_iota(jnp.int32, (G, NB), 0)
    og = (bb == gi).astype(jnp.float32)
    sums_ref[...] += jnp.dot(og, outb, preferred_element_type=jnp.float32)
    counts_ref[...] += jnp.dot(og, jnp.ones_like(outb),
                               preferred_element_type=jnp.float32)

    @pl.when(i == pl.num_programs(0) - 1)
    def _():
        pooled = sums_ref[...] / jnp.maximum(counts_ref[...], 1.0)
        o_ref[...] = jnp.dot(jnp.maximum(pooled, 0.0), wfc_ref[...],
                             preferred_element_type=jnp.float32) + bfc_ref[...]


def _vgather(v, idx):
    dnums = lax.GatherDimensionNumbers(
        offset_dims=(), collapsed_slice_dims=(0,), start_index_map=(0,))
    return lax.gather(v, idx[:, None], dnums, slice_sizes=(1,),
                      mode=lax.GatherScatterMode.PROMISE_IN_BOUNDS)


def _sc_gather_combine(xp, src2d, hpair):
    mesh = plsc.VectorSubcoreMesh(core_axis_name="c", subcore_axis_name="s")

    @functools.partial(
        pl.kernel,
        out_type=jax.ShapeDtypeStruct((E_PAD // 2, 16), jnp.float32),
        mesh=mesh,
        scratch_types=[
            pltpu.VMEM((2, 4, 128), jnp.int32),
            pltpu.VMEM((2, GCH, XPW), jnp.float32),
            pltpu.VMEM((2, GCH // 2, 16), jnp.float32),
            pltpu.VMEM((2, GCH // 2, 16), jnp.float32),
            pltpu.SemaphoreType.DMA,
            pltpu.SemaphoreType.DMA,
            pltpu.SemaphoreType.DMA,
            pltpu.SemaphoreType.DMA,
            pltpu.SemaphoreType.DMA,
        ],
        compiler_params=pltpu.CompilerParams(use_tc_tiling_on_sc=False),
    )
    def gk(xp_hbm, src_hbm, hp_hbm, out_hbm, idx_v, rows_v, h_v, msg_v,
           gsem0, gsem1, hsem0, hsem1, osem):
        c = lax.axis_index("c")
        s = lax.axis_index("s")
        wid = s * NC + c
        gsems = (gsem0, gsem1)
        hsems = (hsem0, hsem1)

        def issue(ci, b):
            pltpu.sync_copy(
                src_hbm.at[pl.ds(wid * (EPW // 128) + ci * 4, 4)],
                idx_v.at[b])
            cps = [pltpu.async_copy(xp_hbm.at[idx_v.at[b].at[j]],
                                    rows_v.at[b].at[pl.ds(j * 128, 128)],
                                    gsems[b])
                   for j in range(4)]
            hcp = pltpu.async_copy(
                hp_hbm.at[pl.ds(wid * (EPW // 2) + ci * (GCH // 2),
                                GCH // 2)],
                h_v.at[b], hsems[b])
            return cps, hcp

        iota = lax.broadcasted_iota(jnp.int32, (16,), 0)
        swp = iota ^ 8
        lo = iota < 8

        def compute(b):
            rows = rows_v.at[b]
            hrows = h_v.at[b]
            mrows = msg_v.at[b]

            def pair(p, carry):
                hv = hrows[p, :]
                acc_a = jnp.zeros((16,), jnp.float32)
                acc_b = jnp.zeros((16,), jnp.float32)
                for r in range(4):
                    ia = jnp.where(lo, r, r + 4)
                    va = rows[2 * p, pl.ds(16 * r, 16)]
                    vb = rows[2 * p + 1, pl.ds(16 * r, 16)]
                    acc_a = acc_a + va * _vgather(hv, ia)
                    acc_b = acc_b + vb * _vgather(hv, ia + 8)
                macc_a = acc_a + _vgather(acc_a, swp)
                macc_b = acc_b + _vgather(acc_b, swp)
                bias_a = rows[2 * p, pl.ds(64, 16)]
                bias_b = rows[2 * p + 1, pl.ds(64, 16)]
                bias = jnp.where(lo, bias_a, _vgather(bias_b, swp))
                mrows[p, :] = jnp.where(lo, macc_a, macc_b) + bias
                return carry

            lax.fori_loop(0, GCH // 2, pair, 0)

        pending = issue(0, 0)
        out_cp = None
        for ci in range(NGCH):
            b = ci % 2
            cps, hcp = pending
            if ci + 1 < NGCH:
                pending = issue(ci + 1, 1 - b)
            for cp in cps:
                cp.wait()
            hcp.wait()
            if out_cp is not None:
                out_cp.wait()
            compute(b)
            out_cp = pltpu.async_copy(
                msg_v.at[b],
                out_hbm.at[pl.ds(wid * (EPW // 2) + ci * (GCH // 2),
                                 GCH // 2)],
                osem)
        out_cp.wait()

    return gk(xp, src2d, hpair)


def _sc_scatter(msg, dst2d, zrows):
    mesh = plsc.VectorSubcoreMesh(core_axis_name="c", subcore_axis_name="s")

    @functools.partial(
        pl.kernel,
        out_type=jax.ShapeDtypeStruct((NC * N_ACC, LAT), jnp.float32),
        mesh=mesh,
        scratch_types=[
            pltpu.VMEM((8, 128), jnp.int32),
            pltpu.VMEM((SCH, LAT), jnp.float32),
            pltpu.VMEM_SHARED((N_ACC, LAT), jnp.float32),
        ],
        compiler_params=pltpu.CompilerParams(use_tc_tiling_on_sc=False),
    )
    def sk(msg_hbm, dst_hbm, z_hbm, out_hbm, idx_v, rows_v, acc):
        c = lax.axis_index("c")
        s = lax.axis_index("s")
        wid = s * NC + c
        pltpu.sync_copy(z_hbm.at[pl.ds(s * ROWS_PER_TILE, ROWS_PER_TILE)],
                        acc.at[pl.ds(s * ROWS_PER_TILE, ROWS_PER_TILE)])
        plsc.subcore_barrier()

        def body(ci, carry):
            base = wid * EPW + ci * SCH
            pltpu.sync_copy(dst_hbm.at[pl.ds(wid * (EPW // 128) + ci * 8, 8)],
                            idx_v)
            pltpu.sync_copy(msg_hbm.at[pl.ds(base, SCH)], rows_v)
            for j in range(8):
                pltpu.sync_copy(rows_v.at[pl.ds(j * 128, 128)],
                                acc.at[idx_v.at[j]], add=True)
            return carry

        lax.fori_loop(0, NSCH, body, 0)
        plsc.subcore_barrier()
        pltpu.sync_copy(
            acc.at[pl.ds(s * ROWS_PER_TILE, ROWS_PER_TILE)],
            out_hbm.at[pl.ds(c * N_ACC + s * ROWS_PER_TILE, ROWS_PER_TILE)])

    return sk(msg, dst2d, zrows)


def kernel(x, edge_index, edge_attr, batch, W1, b1, W2, b2,
           W_root, b_root, Wfc, bfc):
    src = edge_index[0].astype(jnp.int32)
    dst = edge_index[1].astype(jnp.int32)

    w2r = W2.reshape(LAT, DF, LAT).transpose(1, 0, 2).reshape(DF, LAT * LAT)
    cc = jnp.arange(LAT * LAT)
    rr, ll = cc // 16, cc % 16
    kk = jnp.where(ll < 8, rr, rr + 4)
    perm = kk * LAT + (ll % 8)
    w2p = w2r[:, perm]
    wcat = jnp.concatenate([w2p, b2.reshape(DF, LAT), W_root], axis=1)

    pad = E_PAD - E
    src_p = jnp.concatenate([src, jnp.zeros((pad,), jnp.int32)])
    dst_p = jnp.concatenate([dst, jnp.full((pad,), N, jnp.int32)])
    ea_p = jnp.concatenate([edge_attr, jnp.zeros((pad, DE), jnp.float32)])
    src2d = src_p.reshape(E_PAD // 128, 128)
    dst2d = dst_p.reshape(E_PAD // 128, 128)
    zrows = jnp.zeros((N_ACC, LAT), jnp.float32)
    batch3 = batch.astype(jnp.int32).reshape(N // NB, 1, NB)

    ea2 = ea_p.reshape(E_PAD // 2, 2 * DE)
    w1p = jnp.concatenate([
        jnp.concatenate([W1, jnp.zeros((DE, LAT), jnp.float32)], axis=1),
        jnp.concatenate([jnp.zeros((DE, LAT), jnp.float32), W1], axis=1),
    ], axis=0)
    b1p = jnp.concatenate([b1, b1]).reshape(1, 16)
    heb = (E_PAD // 2) // (N // NB)
    xp, hpair = pl.pallas_call(
        _prep_body,
        grid=(N // NB,),
        in_specs=[
            pl.BlockSpec((NB, DF), lambda i: (i, 0)),
            pl.BlockSpec((DF, XPW), lambda i: (0, 0)),
            pl.BlockSpec((heb, 2 * DE), lambda i: (i, 0)),
            pl.BlockSpec((2 * DE, 16), lambda i: (0, 0)),
            pl.BlockSpec((1, 16), lambda i: (0, 0)),
        ],
        out_specs=[
            pl.BlockSpec((NB, XPW), lambda i: (i, 0)),
            pl.BlockSpec((heb, 16), lambda i: (i, 0)),
        ],
        out_shape=[
            jax.ShapeDtypeStruct((N, XPW), jnp.float32),
            jax.ShapeDtypeStruct((E_PAD // 2, 16), jnp.float32),
        ],
    )(x, wcat, ea2, w1p, b1p)

    msg2 = _sc_gather_combine(xp, src2d, hpair)
    msg = msg2.reshape(E_PAD, LAT)

    p01 = _sc_scatter(msg, dst2d, zrows)
    poff = N_ACC // NB

    res = pl.pallas_call(
        _post_body,
        grid=(N // NB,),
        in_specs=[
            pl.BlockSpec((NB, LAT), lambda i: (i, 0)),
            pl.BlockSpec((NB, LAT), lambda i: (i + poff, 0)),
            pl.BlockSpec((NB, XPW), lambda i: (i, 0)),
            pl.BlockSpec((1, 1, NB), lambda i: (i, 0, 0)),
            pl.BlockSpec((1, LAT), lambda i: (0, 0)),
            pl.BlockSpec((LAT, EMB), lambda i: (0, 0)),
            pl.BlockSpec((1, EMB), lambda i: (0, 0)),
        ],
        out_specs=pl.BlockSpec((G, EMB), lambda i: (0, 0)),
        out_shape=jax.ShapeDtypeStruct((G, EMB), jnp.float32),
        scratch_shapes=[
            pltpu.VMEM((G, LAT), jnp.float32),
            pltpu.VMEM((G, LAT), jnp.float32),
        ],
    )(p01, p01, xp, batch3, b_root.reshape(1, LAT), Wfc, bfc.reshape(1, EMB))

    return res


# speedup vs baseline: 3.7985x; 1.1235x over previous
import functools

import jax
import jax.numpy as jnp
from jax import lax
from jax.experimental import pallas as pl
from jax.experimental.pallas import tpu as pltpu
from jax.experimental.pallas import tpu_sc as plsc

N = 10000
E = 160000
DF = 128
DE = 4
LAT = 8
EMB = 64
G = 64

NC = 2
NS = 16
NW = NC * NS
EPW = 5120
E_PAD = NW * EPW
GCH = 512
NGCH = EPW // GCH
SCH = 1024
NSCH = EPW // SCH
N_ACC = 16000
ROWS_PER_TILE = N_ACC // NS

XPW = 80

NB = 1000


def _prep_body(x_ref, w_ref, ea2_ref, w1p_ref, b1p_ref,
               xp_ref, root_ref, h2_ref):
    xpb = jnp.dot(x_ref[...], w_ref[...], preferred_element_type=jnp.float32)
    xp_ref[...] = xpb
    root_ref[...] = xpb[:, 72:80]
    h2_ref[...] = jnp.maximum(
        jnp.dot(ea2_ref[...], w1p_ref[...],
                preferred_element_type=jnp.float32) + b1p_ref[...], 0.0)


def _post_body(p0_ref, p1_ref, root_ref, b3_ref, broot_ref, wfc_ref, bfc_ref,
               o_ref, sums_ref, counts_ref):
    i = pl.program_id(0)

    @pl.when(i == 0)
    def _():
        sums_ref[...] = jnp.zeros_like(sums_ref)
        counts_ref[...] = jnp.zeros_like(counts_ref)

    outb = jnp.maximum(
        p0_ref[...] + p1_ref[...] + root_ref[...] + broot_ref[...], 0.0)
    bb = b3_ref[0]
    gi = lax.broadcasted_iota(jnp.int32, (G, NB), 0)
    og = (bb == gi).astype(jnp.float32)
    sums_ref[...] += jnp.dot(og, outb, preferred_element_type=jnp.float32)
    counts_ref[...] += jnp.dot(og, jnp.ones_like(outb),
                               preferred_element_type=jnp.float32)

    @pl.when(i == pl.num_programs(0) - 1)
    def _():
        pooled = sums_ref[...] / jnp.maximum(counts_ref[...], 1.0)
        o_ref[...] = jnp.dot(jnp.maximum(pooled, 0.0), wfc_ref[...],
                             preferred_element_type=jnp.float32) + bfc_ref[...]


def _vgather(v, idx):
    dnums = lax.GatherDimensionNumbers(
        offset_dims=(), collapsed_slice_dims=(0,), start_index_map=(0,))
    return lax.gather(v, idx[:, None], dnums, slice_sizes=(1,),
                      mode=lax.GatherScatterMode.PROMISE_IN_BOUNDS)


def _sc_gather_combine(xp, src2d, hpair):
    mesh = plsc.VectorSubcoreMesh(core_axis_name="c", subcore_axis_name="s")

    @functools.partial(
        pl.kernel,
        out_type=jax.ShapeDtypeStruct((E_PAD // 2, 16), jnp.float32),
        mesh=mesh,
        scratch_types=[
            pltpu.VMEM((2, 4, 128), jnp.int32),
            pltpu.VMEM((2, GCH, XPW), jnp.float32),
            pltpu.VMEM((2, GCH // 2, 16), jnp.float32),
            pltpu.VMEM((2, GCH // 2, 16), jnp.float32),
            pltpu.SemaphoreType.DMA,
            pltpu.SemaphoreType.DMA,
            pltpu.SemaphoreType.DMA,
            pltpu.SemaphoreType.DMA,
            pltpu.SemaphoreType.DMA,
        ],
        compiler_params=pltpu.CompilerParams(use_tc_tiling_on_sc=False),
    )
    def gk(xp_hbm, src_hbm, hp_hbm, out_hbm, idx_v, rows_v, h_v, msg_v,
           gsem0, gsem1, hsem0, hsem1, osem):
        c = lax.axis_index("c")
        s = lax.axis_index("s")
        wid = s * NC + c
        gsems = (gsem0, gsem1)
        hsems = (hsem0, hsem1)

        def issue(ci, b):
            pltpu.sync_copy(
                src_hbm.at[pl.ds(wid * (EPW // 128) + ci * 4, 4)],
                idx_v.at[b])
            cps = [pltpu.async_copy(xp_hbm.at[idx_v.at[b].at[j]],
                                    rows_v.at[b].at[pl.ds(j * 128, 128)],
                                    gsems[b])
                   for j in range(4)]
            hcp = pltpu.async_copy(
                hp_hbm.at[pl.ds(wid * (EPW // 2) + ci * (GCH // 2),
                                GCH // 2)],
                h_v.at[b], hsems[b])
            return cps, hcp

        iota = lax.broadcasted_iota(jnp.int32, (16,), 0)
        swp = iota ^ 8
        lo = iota < 8

        def compute(b):
            rows = rows_v.at[b]
            hrows = h_v.at[b]
            mrows = msg_v.at[b]

            def pair(p, carry):
                hv = hrows[p, :]
                acc_a = jnp.zeros((16,), jnp.float32)
                acc_b = jnp.zeros((16,), jnp.float32)
                for r in range(4):
                    ia = jnp.where(lo, r, r + 4)
                    va = rows[2 * p, pl.ds(16 * r, 16)]
                    vb = rows[2 * p + 1, pl.ds(16 * r, 16)]
                    acc_a = acc_a + va * _vgather(hv, ia)
                    acc_b = acc_b + vb * _vgather(hv, ia + 8)
                macc_a = acc_a + _vgather(acc_a, swp)
                macc_b = acc_b + _vgather(acc_b, swp)
                bias_a = rows[2 * p, pl.ds(64, 16)]
                bias_b = rows[2 * p + 1, pl.ds(64, 16)]
                bias = jnp.where(lo, bias_a, _vgather(bias_b, swp))
                mrows[p, :] = jnp.where(lo, macc_a, macc_b) + bias
                return carry

            lax.fori_loop(0, GCH // 2, pair, 0)

        pending = issue(0, 0)
        out_cp = None
        for ci in range(NGCH):
            b = ci % 2
            cps, hcp = pending
            if ci + 1 < NGCH:
                pending = issue(ci + 1, 1 - b)
            for cp in cps:
                cp.wait()
            hcp.wait()
            if out_cp is not None:
                out_cp.wait()
            compute(b)
            out_cp = pltpu.async_copy(
                msg_v.at[b],
                out_hbm.at[pl.ds(wid * (EPW // 2) + ci * (GCH // 2),
                                 GCH // 2)],
                osem)
        out_cp.wait()

    return gk(xp, src2d, hpair)


def _sc_scatter(msg, dst2d, zrows):
    mesh = plsc.VectorSubcoreMesh(core_axis_name="c", subcore_axis_name="s")

    @functools.partial(
        pl.kernel,
        out_type=jax.ShapeDtypeStruct((NC * N_ACC, LAT), jnp.float32),
        mesh=mesh,
        scratch_types=[
            pltpu.VMEM((8, 128), jnp.int32),
            pltpu.VMEM((SCH, LAT), jnp.float32),
            pltpu.VMEM_SHARED((N_ACC, LAT), jnp.float32),
        ],
        compiler_params=pltpu.CompilerParams(use_tc_tiling_on_sc=False),
    )
    def sk(msg_hbm, dst_hbm, z_hbm, out_hbm, idx_v, rows_v, acc):
        c = lax.axis_index("c")
        s = lax.axis_index("s")
        wid = s * NC + c
        pltpu.sync_copy(z_hbm.at[pl.ds(s * ROWS_PER_TILE, ROWS_PER_TILE)],
                        acc.at[pl.ds(s * ROWS_PER_TILE, ROWS_PER_TILE)])
        plsc.subcore_barrier()

        def body(ci, carry):
            base = wid * EPW + ci * SCH
            pltpu.sync_copy(dst_hbm.at[pl.ds(wid * (EPW // 128) + ci * 8, 8)],
                            idx_v)
            pltpu.sync_copy(msg_hbm.at[pl.ds(base, SCH)], rows_v)
            for j in range(8):
                pltpu.sync_copy(rows_v.at[pl.ds(j * 128, 128)],
                                acc.at[idx_v.at[j]], add=True)
            return carry

        lax.fori_loop(0, NSCH, body, 0)
        plsc.subcore_barrier()
        pltpu.sync_copy(
            acc.at[pl.ds(s * ROWS_PER_TILE, ROWS_PER_TILE)],
            out_hbm.at[pl.ds(c * N_ACC + s * ROWS_PER_TILE, ROWS_PER_TILE)])

    return sk(msg, dst2d, zrows)


def kernel(x, edge_index, edge_attr, batch, W1, b1, W2, b2,
           W_root, b_root, Wfc, bfc):
    w2r = W2.reshape(LAT, DF, LAT).transpose(1, 0, 2).reshape(DF, LAT * LAT)
    cc = jnp.arange(LAT * LAT)
    rr, ll = cc // 16, cc % 16
    kk = jnp.where(ll < 8, rr, rr + 4)
    perm = kk * LAT + (ll % 8)
    w2p = w2r[:, perm]
    wcat = jnp.concatenate([w2p, b2.reshape(DF, LAT), W_root], axis=1)

    pad = E_PAD - E
    src = edge_index[0].astype(jnp.int32)
    dst = edge_index[1].astype(jnp.int32)
    src_p = jnp.concatenate([src, jnp.zeros((pad,), jnp.int32)])
    dst_p = jnp.concatenate([dst, jnp.full((pad,), N, jnp.int32)])
    src2d = src_p.reshape(E_PAD // 128, 128)
    dst2d = dst_p.reshape(E_PAD // 128, 128)
    zrows = jnp.zeros((N_ACC, LAT), jnp.float32)
    batch3 = batch.astype(jnp.int32).reshape(N // NB, 1, NB)

    ea2 = edge_attr.reshape(E // 2, 2 * DE)
    w1p = jnp.concatenate([
        jnp.concatenate([W1, jnp.zeros((DE, LAT), jnp.float32)], axis=1),
        jnp.concatenate([jnp.zeros((DE, LAT), jnp.float32), W1], axis=1),
    ], axis=0)
    b1p = jnp.concatenate([b1, b1]).reshape(1, 16)

    heb = (E // 2) // (N // NB)
    xp, root, hpair = pl.pallas_call(
        _prep_body,
        grid=(N // NB,),
        in_specs=[
            pl.BlockSpec((NB, DF), lambda i: (i, 0)),
            pl.BlockSpec((DF, XPW), lambda i: (0, 0)),
            pl.BlockSpec((heb, 2 * DE), lambda i: (i, 0)),
            pl.BlockSpec((2 * DE, 16), lambda i: (0, 0)),
            pl.BlockSpec((1, 16), lambda i: (0, 0)),
        ],
        out_specs=[
            pl.BlockSpec((NB, XPW), lambda i: (i, 0)),
            pl.BlockSpec((NB, LAT), lambda i: (i, 0)),
            pl.BlockSpec((heb, 16), lambda i: (i, 0)),
        ],
        out_shape=[
            jax.ShapeDtypeStruct((N, XPW), jnp.float32),
            jax.ShapeDtypeStruct((N, LAT), jnp.float32),
            jax.ShapeDtypeStruct((E_PAD // 2, 16), jnp.float32),
        ],
    )(x, wcat, ea2, w1p, b1p)

    msg2 = _sc_gather_combine(xp, src2d, hpair)
    msg = msg2.reshape(E_PAD, LAT)

    p01 = _sc_scatter(msg, dst2d, zrows)
    poff = N_ACC // NB

    res = pl.pallas_call(
        _post_body,
        grid=(N // NB,),
        in_specs=[
            pl.BlockSpec((NB, LAT), lambda i: (i, 0)),
            pl.BlockSpec((NB, LAT), lambda i: (i + poff, 0)),
            pl.BlockSpec((NB, LAT), lambda i: (i, 0)),
            pl.BlockSpec((1, 1, NB), lambda i: (i, 0, 0)),
            pl.BlockSpec((1, LAT), lambda i: (0, 0)),
            pl.BlockSpec((LAT, EMB), lambda i: (0, 0)),
            pl.BlockSpec((1, EMB), lambda i: (0, 0)),
        ],
        out_specs=pl.BlockSpec((G, EMB), lambda i: (0, 0)),
        out_shape=jax.ShapeDtypeStruct((G, EMB), jnp.float32),
        scratch_shapes=[
            pltpu.VMEM((G, LAT), jnp.float32),
            pltpu.VMEM((G, LAT), jnp.float32),
        ],
    )(p01, p01, root, batch3, b_root.reshape(1, LAT), Wfc, bfc.reshape(1, EMB))

    return res
